# baseline (speedup 1.0000x reference)
import functools

import jax
import jax.numpy as jnp
from jax import lax
from jax.experimental import pallas as pl
from jax.experimental.pallas import tpu as pltpu
from jax.experimental.pallas import tpu_sc as plsc

N = 10000
H = 128
E = 320000
EL = 50000

NPAD = 10240
NC, NS = 2, 16
NW = NC * NS
SK = 40
K1 = 80
E1 = NW * K1 * 128
K2 = 160
E2 = NS * K2 * 128
ROWB = NPAD // NS

_mesh = plsc.VectorSubcoreMesh(core_axis_name="c", subcore_axis_name="s")


def _zero_buf(buf):
    def zb(t, c):
        buf[t >> 3, pl.ds((t & 7) * 16, 16)] = jnp.zeros((16,), jnp.float32)
        return c
    lax.fori_loop(0, 1024, zb, 0)


def _run_pipeline(tab_hbm, ghw, shw, gidx, sidx, buf0, buf1,
                  gsem0, gsem1, ssem0, ssem1, cntsem,
                  acc, nstages, counts=None):
    def gwait(jj, buf, gsem):
        pltpu.make_async_copy(tab_hbm.at[gidx.at[jj]], buf, gsem).wait()

    def sissue(jj, buf, ssem):
        pltpu.async_copy(buf, acc.at[sidx.at[jj]], ssem, add=True)
        if counts is not None:
            ones_buf, cpacc, caacc = counts
            pltpu.async_copy(ones_buf, cpacc.at[gidx.at[jj]], cntsem,
                             add=True)
            pltpu.async_copy(ones_buf, caacc.at[sidx.at[jj]], cntsem,
                             add=True)

    def swait(jj, buf, ssem):
        pltpu.make_async_copy(buf, acc.at[sidx.at[jj]], ssem).wait()

    def cntwait(jj):
        ones_buf, cpacc, caacc = counts
        pltpu.make_async_copy(ones_buf, cpacc.at[gidx.at[jj]], cntsem).wait()
        pltpu.make_async_copy(ones_buf, caacc.at[sidx.at[jj]], cntsem).wait()

    def stage(s, c):
        pltpu.sync_copy(ghw.at[pl.ds(s * SK, SK)], gidx)
        pltpu.sync_copy(shw.at[pl.ds(s * SK, SK)], sidx)
        pltpu.async_copy(tab_hbm.at[gidx.at[0]], buf0, gsem0)

        def pair(t, c2):
            jj0 = 2 * t
            jj1 = 2 * t + 1
            gwait(jj0, buf0, gsem0)
            sissue(jj0, buf0, ssem0)

            @pl.when(jj0 >= 1)
            def _():
                swait(jj0 - 1, buf1, ssem1)
            pltpu.async_copy(tab_hbm.at[gidx.at[jj1]], buf1, gsem1)
            if counts is not None:
                @pl.when(jj0 >= 2)
                def _():
                    cntwait(jj0 - 2)
            gwait(jj1, buf1, gsem1)
            sissue(jj1, buf1, ssem1)
            swait(jj0, buf0, ssem0)

            @pl.when(jj1 + 1 < SK)
            def _():
                pltpu.async_copy(tab_hbm.at[gidx.at[jj1 + 1]], buf0, gsem0)
            if counts is not None:
                @pl.when(jj1 >= 2)
                def _():
                    cntwait(jj1 - 2)
            return c2

        c = lax.fori_loop(0, SK // 2, pair, c)
        swait(SK - 1, buf1, ssem1)
        if counts is not None:
            cntwait(SK - 2)
            cntwait(SK - 1)
        return c

    lax.fori_loop(0, nstages, stage, 0)


def _seg1_body(x_hbm, g_hbm, s_hbm, out_s, out_cp, out_ca,
               gidx, sidx, buf0, buf1, ones_buf, acc, cpacc, caacc,
               gsem0, gsem1, ssem0, ssem1, cntsem):
    cid = lax.axis_index("c")
    sid = lax.axis_index("s")
    wid = cid * NS + sid
    _zero_buf(buf0)
    for v in range(8):
        ones_buf[pl.ds(v * 16, 16)] = jnp.ones((16,), jnp.float32)
    for k in range(ROWB // 128):
        pltpu.sync_copy(buf0, acc.at[pl.ds(sid * ROWB + k * 128, 128)])
        pltpu.sync_copy(buf0.at[0], cpacc.at[pl.ds(sid * ROWB + k * 128, 128)])
        pltpu.sync_copy(buf0.at[0], caacc.at[pl.ds(sid * ROWB + k * 128, 128)])
    plsc.subcore_barrier()

    _run_pipeline(x_hbm, g_hbm.at[wid], s_hbm.at[wid], gidx, sidx,
                  buf0, buf1, gsem0, gsem1, ssem0, ssem1, cntsem,
                  acc, K1 // SK, counts=(ones_buf, cpacc, caacc))

    plsc.subcore_barrier()
    pltpu.sync_copy(acc.at[pl.ds(sid * ROWB, ROWB)],
                    out_s.at[cid, pl.ds(sid * ROWB, ROWB)])
    pltpu.sync_copy(cpacc.at[pl.ds(sid * ROWB, ROWB)],
                    out_cp.at[cid, pl.ds(sid * ROWB, ROWB)])
    pltpu.sync_copy(caacc.at[pl.ds(sid * ROWB, ROWB)],
                    out_ca.at[cid, pl.ds(sid * ROWB, ROWB)])


def _seg2_body(tab_hbm, g_hbm, s_hbm, out_s, gidx, sidx, buf0, buf1, acc,
               gsem0, gsem1, ssem0, ssem1):
    cid = lax.axis_index("c")
    sid = lax.axis_index("s")
    _zero_buf(buf0)
    for k in range(ROWB // 128):
        pltpu.sync_copy(buf0, acc.at[pl.ds(sid * ROWB + k * 128, 128)])
    plsc.subcore_barrier()

    _run_pipeline(tab_hbm, g_hbm.at[cid, sid], s_hbm.at[cid, sid], gidx, sidx,
                  buf0, buf1, gsem0, gsem1, ssem0, ssem1, None,
                  acc, K2 // SK)

    plsc.subcore_barrier()
    pltpu.sync_copy(acc.at[pl.ds(sid * ROWB, ROWB)],
                    out_s.at[cid, pl.ds(sid * ROWB, ROWB)])


_seg1 = pl.kernel(
    _seg1_body,
    out_type=[jax.ShapeDtypeStruct((NC, NPAD, H), jnp.float32),
              jax.ShapeDtypeStruct((NC, NPAD), jnp.float32),
              jax.ShapeDtypeStruct((NC, NPAD), jnp.float32)],
    mesh=_mesh,
    scratch_types=[
        pltpu.VMEM((SK, 128), jnp.int32),
        pltpu.VMEM((SK, 128), jnp.int32),
        pltpu.VMEM((128, H), jnp.float32),
        pltpu.VMEM((128, H), jnp.float32),
        pltpu.VMEM((128,), jnp.float32),
        pltpu.VMEM_SHARED((NPAD, H), jnp.float32),
        pltpu.VMEM_SHARED((NPAD,), jnp.float32),
        pltpu.VMEM_SHARED((NPAD,), jnp.float32),
        pltpu.SemaphoreType.DMA,
        pltpu.SemaphoreType.DMA,
        pltpu.SemaphoreType.DMA,
        pltpu.SemaphoreType.DMA,
        pltpu.SemaphoreType.DMA,
    ],
)

_seg2 = pl.kernel(
    _seg2_body,
    out_type=[jax.ShapeDtypeStruct((NC, NPAD, H), jnp.float32)],
    mesh=_mesh,
    scratch_types=[
        pltpu.VMEM((SK, 128), jnp.int32),
        pltpu.VMEM((SK, 128), jnp.int32),
        pltpu.VMEM((128, H), jnp.float32),
        pltpu.VMEM((128, H), jnp.float32),
        pltpu.VMEM_SHARED((NPAD, H), jnp.float32),
        pltpu.SemaphoreType.DMA,
        pltpu.SemaphoreType.DMA,
        pltpu.SemaphoreType.DMA,
        pltpu.SemaphoreType.DMA,
    ],
)


ROW_BLK = 2048


def _dense1_body(cnt_p_ref, cnt_a_ref, xp_ref, s1_ref, w1wpl_ref, w1wpr_ref,
                 w1rwl_ref, w1rwr_ref, b1wp_ref, b1rw_ref, out_ref):
    cnt_p = cnt_p_ref[...]
    ind_p = (cnt_p > 0.0).astype(jnp.float32)
    w1sum = jnp.sum(w1wpl_ref[...], axis=0, keepdims=True)
    hp = ind_p * w1sum + jnp.dot(xp_ref[...], w1wpr_ref[...],
                                 preferred_element_type=jnp.float32)
    out_ref[1] = jnp.maximum(hp + b1wp_ref[...], 0.0)
    cnt_a = jnp.maximum(cnt_a_ref[...], 1.0)
    m_a = (s1_ref[0] + s1_ref[1]) / cnt_a
    c = jnp.sum(w1rwr_ref[...], axis=0, keepdims=True) + b1rw_ref[...]
    ha = jnp.dot(m_a, w1rwl_ref[...], preferred_element_type=jnp.float32) + c
    out_ref[0] = jnp.maximum(ha, 0.0)


def _dense2_body(cnt_p_ref, cnt_a_ref, h_ref, s2_ref, w2wpl_ref, w2wpr_ref,
                 w2rwl_ref, w2rwr_ref, b2wp_ref, b2rw_ref, out_ref):
    cnt_p = jnp.maximum(cnt_p_ref[...], 1.0)
    cnt_a = jnp.maximum(cnt_a_ref[...], 1.0)
    out_ref[1] = (jnp.dot(s2_ref[0] / cnt_p, w2wpl_ref[...],
                          preferred_element_type=jnp.float32)
                  + jnp.dot(h_ref[1], w2wpr_ref[...],
                            preferred_element_type=jnp.float32)
                  + b2wp_ref[...])
    out_ref[0] = (jnp.dot(s2_ref[1] / cnt_a, w2rwl_ref[...],
                          preferred_element_type=jnp.float32)
                  + jnp.dot(h_ref[0], w2rwr_ref[...],
                            preferred_element_type=jnp.float32)
                  + b2rw_ref[...])


def _cls_body(ga_ref, gp_ref, out_ref):
    out_ref[...] = jnp.sum(ga_ref[...] * gp_ref[...], axis=-1)


def _row_spec():
    return pl.BlockSpec((ROW_BLK, H), lambda i: (i, 0))


def _pair_spec():
    return pl.BlockSpec((2, ROW_BLK, H), lambda i: (0, i, 0))


def _full_spec():
    return pl.BlockSpec((H, H), lambda i: (0, 0))


def _bias_spec():
    return pl.BlockSpec((1, H), lambda i: (0, 0))


def _cnt_spec():
    return pl.BlockSpec((ROW_BLK, 1), lambda i: (i, 0))


def kernel(x_paper, x_author, edge_index_writes, edge_index_rev,
           edge_label_index, W1_wp_l, W1_wp_r, W1_rw_l, W1_rw_r, W2_wp_l,
           W2_wp_r, W2_rw_l, W2_rw_r, b1_wp, b1_rw, b2_wp, b2_rw):
    ew0 = edge_index_writes[0].astype(jnp.int32)
    ew1 = edge_index_writes[1].astype(jnp.int32)
    x_pad = jnp.pad(x_paper, ((0, NPAD - N), (0, 0)))

    pad1 = E1 - E
    g1 = jnp.pad(ew1, (0, pad1), constant_values=N).reshape(NW, K1, 128)
    s1 = jnp.pad(ew0, (0, pad1), constant_values=N).reshape(NW, K1, 128)
    s1p, cp_p, ca_p = _seg1(x_pad, g1, s1)
    cnt_p = (cp_p[0] + cp_p[1]).reshape(NPAD, 1)
    cnt_a = (ca_p[0] + ca_p[1]).reshape(NPAD, 1)

    grid = (NPAD // ROW_BLK,)
    hcat = pl.pallas_call(
        _dense1_body,
        grid=grid,
        in_specs=[_cnt_spec(), _cnt_spec(), _row_spec(), _pair_spec(),
                  _full_spec(), _full_spec(), _full_spec(), _full_spec(),
                  _bias_spec(), _bias_spec()],
        out_specs=_pair_spec(),
        out_shape=jax.ShapeDtypeStruct((2, NPAD, H), jnp.float32),
    )(cnt_p, cnt_a, x_pad, s1p, W1_wp_l, W1_wp_r, W1_rw_l, W1_rw_r,
      b1_wp[None, :], b1_rw[None, :])

    pad2 = E2 - E
    g2 = jnp.stack([
        jnp.pad(ew0, (0, pad2), constant_values=N),
        jnp.pad(ew1, (0, pad2), constant_values=N) + NPAD,
    ]).reshape(NC, NS, K2, 128)
    s2 = jnp.stack([
        jnp.pad(ew1, (0, pad2), constant_values=N),
        jnp.pad(ew0, (0, pad2), constant_values=N),
    ]).reshape(NC, NS, K2, 128)
    (s2out,) = _seg2(hcat.reshape(NC * NPAD, H), g2, s2)

    hcat2 = pl.pallas_call(
        _dense2_body,
        grid=grid,
        in_specs=[_cnt_spec(), _cnt_spec(), _pair_spec(), _pair_spec(),
                  _full_spec(), _full_spec(), _full_spec(), _full_spec(),
                  _bias_spec(), _bias_spec()],
        out_specs=_pair_spec(),
        out_shape=jax.ShapeDtypeStruct((2, NPAD, H), jnp.float32),
    )(cnt_p, cnt_a, hcat, s2out, W2_wp_l, W2_wp_r, W2_rw_l, W2_rw_r,
      b2_wp[None, :], b2_rw[None, :])

    ga = jnp.take(hcat2[0], edge_label_index[0], axis=0)
    gp = jnp.take(hcat2[1], edge_label_index[1], axis=0)
    EL_PAD = 50176
    ga = jnp.pad(ga, ((0, EL_PAD - EL), (0, 0)))
    gp = jnp.pad(gp, ((0, EL_PAD - EL), (0, 0)))
    CLS_BLK = 1024
    out = pl.pallas_call(
        _cls_body,
        grid=(EL_PAD // CLS_BLK,),
        in_specs=[pl.BlockSpec((CLS_BLK, H), lambda i: (i, 0))] * 2,
        out_specs=pl.BlockSpec((CLS_BLK,), lambda i: (i,)),
        out_shape=jax.ShapeDtypeStruct((EL_PAD,), jnp.float32),
    )(ga, gp)
    return out[:EL]


# speedup vs baseline: 1.0905x; 1.0905x over previous
import functools

import jax
import jax.numpy as jnp
from jax import lax
from jax.experimental import pallas as pl
from jax.experimental.pallas import tpu as pltpu
from jax.experimental.pallas import tpu_sc as plsc

N = 10000
H = 128
E = 320000
EL = 50000

NPAD = 10240
NC, NS = 2, 16
NW = NC * NS
SK = 40
K1 = 80
E1 = NW * K1 * 128
K2 = 160
E2 = NS * K2 * 128
ROWB = NPAD // NS

_mesh = plsc.VectorSubcoreMesh(core_axis_name="c", subcore_axis_name="s")


def _zero_buf(buf):
    def zb(t, c):
        buf[t >> 3, pl.ds((t & 7) * 16, 16)] = jnp.zeros((16,), jnp.float32)
        return c
    lax.fori_loop(0, 1024, zb, 0)


def _run_pipeline(tab_hbm, ghw, shw, gidx, sidx, buf0, buf1,
                  gsem0, gsem1, ssem0, ssem1, cntsem,
                  acc, nstages, counts=None):
    def stage(s, c):
        pltpu.sync_copy(ghw.at[pl.ds(s * SK, SK)], gidx)
        pltpu.sync_copy(shw.at[pl.ds(s * SK, SK)], sidx)
        pltpu.async_copy(tab_hbm.at[gidx.at[0]], buf0, gsem0)
        pltpu.async_copy(tab_hbm.at[gidx.at[1]], buf1, gsem1)

        def step(jj, buf, sem):
            pltpu.make_async_copy(tab_hbm.at[gidx.at[jj]], buf, sem).wait()
            pltpu.sync_copy(buf, acc.at[sidx.at[jj]], add=True)
            if counts is not None:
                ones_buf, cpacc, caacc = counts
                pltpu.sync_copy(ones_buf, cpacc.at[gidx.at[jj]], add=True)
                pltpu.sync_copy(ones_buf, caacc.at[sidx.at[jj]], add=True)

            @pl.when(jj + 2 < SK)
            def _():
                pltpu.async_copy(tab_hbm.at[gidx.at[jj + 2]], buf, sem)

        def pair(t, c2):
            step(2 * t, buf0, gsem0)
            step(2 * t + 1, buf1, gsem1)
            return c2
        return lax.fori_loop(0, SK // 2, pair, c)

    lax.fori_loop(0, nstages, stage, 0)


def _seg1_body(x_hbm, g_hbm, s_hbm, out_s, out_cp, out_ca,
               gidx, sidx, buf0, buf1, ones_buf, acc, cpacc, caacc,
               gsem0, gsem1, ssem0, ssem1, cntsem):
    cid = lax.axis_index("c")
    sid = lax.axis_index("s")
    wid = cid * NS + sid
    _zero_buf(buf0)
    for v in range(8):
        ones_buf[pl.ds(v * 16, 16)] = jnp.ones((16,), jnp.float32)
    for k in range(ROWB // 128):
        pltpu.sync_copy(buf0, acc.at[pl.ds(sid * ROWB + k * 128, 128)])
        pltpu.sync_copy(buf0.at[0], cpacc.at[pl.ds(sid * ROWB + k * 128, 128)])
        pltpu.sync_copy(buf0.at[0], caacc.at[pl.ds(sid * ROWB + k * 128, 128)])
    plsc.subcore_barrier()

    _run_pipeline(x_hbm, g_hbm.at[wid], s_hbm.at[wid], gidx, sidx,
                  buf0, buf1, gsem0, gsem1, ssem0, ssem1, cntsem,
                  acc, K1 // SK, counts=(ones_buf, cpacc, caacc))

    plsc.subcore_barrier()
    pltpu.sync_copy(acc.at[pl.ds(sid * ROWB, ROWB)],
                    out_s.at[cid, pl.ds(sid * ROWB, ROWB)])
    pltpu.sync_copy(cpacc.at[pl.ds(sid * ROWB, ROWB)],
                    out_cp.at[cid, pl.ds(sid * ROWB, ROWB)])
    pltpu.sync_copy(caacc.at[pl.ds(sid * ROWB, ROWB)],
                    out_ca.at[cid, pl.ds(sid * ROWB, ROWB)])


def _seg2_body(tab_hbm, g_hbm, s_hbm, out_s, gidx, sidx, buf0, buf1, acc,
               gsem0, gsem1, ssem0, ssem1):
    cid = lax.axis_index("c")
    sid = lax.axis_index("s")
    _zero_buf(buf0)
    for k in range(ROWB // 128):
        pltpu.sync_copy(buf0, acc.at[pl.ds(sid * ROWB + k * 128, 128)])
    plsc.subcore_barrier()

    _run_pipeline(tab_hbm, g_hbm.at[cid, sid], s_hbm.at[cid, sid], gidx, sidx,
                  buf0, buf1, gsem0, gsem1, ssem0, ssem1, None,
                  acc, K2 // SK)

    plsc.subcore_barrier()
    pltpu.sync_copy(acc.at[pl.ds(sid * ROWB, ROWB)],
                    out_s.at[cid, pl.ds(sid * ROWB, ROWB)])


_seg1 = pl.kernel(
    _seg1_body,
    out_type=[jax.ShapeDtypeStruct((NC, NPAD, H), jnp.float32),
              jax.ShapeDtypeStruct((NC, NPAD), jnp.float32),
              jax.ShapeDtypeStruct((NC, NPAD), jnp.float32)],
    mesh=_mesh,
    scratch_types=[
        pltpu.VMEM((SK, 128), jnp.int32),
        pltpu.VMEM((SK, 128), jnp.int32),
        pltpu.VMEM((128, H), jnp.float32),
        pltpu.VMEM((128, H), jnp.float32),
        pltpu.VMEM((128,), jnp.float32),
        pltpu.VMEM_SHARED((NPAD, H), jnp.float32),
        pltpu.VMEM_SHARED((NPAD,), jnp.float32),
        pltpu.VMEM_SHARED((NPAD,), jnp.float32),
        pltpu.SemaphoreType.DMA,
        pltpu.SemaphoreType.DMA,
        pltpu.SemaphoreType.DMA,
        pltpu.SemaphoreType.DMA,
        pltpu.SemaphoreType.DMA,
    ],
)

KC = 13
ELP = NW * KC * 128


def _cls_sc_body(tab_hbm, ia_hbm, ib_hbm, out_hbm, iav, ibv,
                 bufA0, bufB0, bufA1, bufB1, outv,
                 semA0, semB0, semA1, semB1):
    cid = lax.axis_index("c")
    sid = lax.axis_index("s")
    wid = cid * NS + sid
    pltpu.sync_copy(ia_hbm.at[wid], iav)
    pltpu.sync_copy(ib_hbm.at[wid], ibv)

    def issue(j, bufA, bufB, semA, semB):
        pltpu.async_copy(tab_hbm.at[iav.at[j]], bufA, semA)
        pltpu.async_copy(tab_hbm.at[ibv.at[j]], bufB, semB)

    def wait(j, bufA, bufB, semA, semB):
        pltpu.make_async_copy(tab_hbm.at[iav.at[j]], bufA, semA).wait()
        pltpu.make_async_copy(tab_hbm.at[ibv.at[j]], bufB, semB).wait()

    def compute(j, bufA, bufB):
        def row(r, c):
            a = bufA[r, pl.ds(0, 16)] * bufB[r, pl.ds(0, 16)]
            for v in range(1, 8):
                a = a + (bufA[r, pl.ds(v * 16, 16)]
                         * bufB[r, pl.ds(v * 16, 16)])
            outv[j, pl.ds(r * 16, 16)] = a
            return c
        lax.fori_loop(0, 128, row, 0)

    issue(0, bufA0, bufB0, semA0, semB0)

    def pair(t, c):
        j0 = 2 * t
        j1 = 2 * t + 1
        wait(j0, bufA0, bufB0, semA0, semB0)
        issue(j0 + 1, bufA1, bufB1, semA1, semB1)
        compute(j0, bufA0, bufB0)
        wait(j1, bufA1, bufB1, semA1, semB1)

        @pl.when(j1 + 1 < KC)
        def _():
            issue(j1 + 1, bufA0, bufB0, semA0, semB0)
        compute(j1, bufA1, bufB1)
        return c
    lax.fori_loop(0, KC // 2, pair, 0)
    wait(KC - 1, bufA0, bufB0, semA0, semB0)
    compute(KC - 1, bufA0, bufB0)
    pltpu.sync_copy(outv, out_hbm.at[wid])


_cls_sc = pl.kernel(
    _cls_sc_body,
    out_type=[jax.ShapeDtypeStruct((NW, KC, 2048), jnp.float32)],
    mesh=_mesh,
    scratch_types=[
        pltpu.VMEM((KC, 128), jnp.int32),
        pltpu.VMEM((KC, 128), jnp.int32),
        pltpu.VMEM((128, H), jnp.float32),
        pltpu.VMEM((128, H), jnp.float32),
        pltpu.VMEM((128, H), jnp.float32),
        pltpu.VMEM((128, H), jnp.float32),
        pltpu.VMEM((KC, 2048), jnp.float32),
        pltpu.SemaphoreType.DMA,
        pltpu.SemaphoreType.DMA,
        pltpu.SemaphoreType.DMA,
        pltpu.SemaphoreType.DMA,
    ],
)


def _lanefold_body(x_ref, out_ref):
    i = lax.broadcasted_iota(jnp.int32, (H, H), 0)
    j = lax.broadcasted_iota(jnp.int32, (H, H), 1)
    m = ((i // 16) == j).astype(jnp.float32)
    out_ref[...] = jnp.dot(x_ref[...], m, preferred_element_type=jnp.float32)


_seg2 = pl.kernel(
    _seg2_body,
    out_type=[jax.ShapeDtypeStruct((NC, NPAD, H), jnp.float32)],
    mesh=_mesh,
    scratch_types=[
        pltpu.VMEM((SK, 128), jnp.int32),
        pltpu.VMEM((SK, 128), jnp.int32),
        pltpu.VMEM((128, H), jnp.float32),
        pltpu.VMEM((128, H), jnp.float32),
        pltpu.VMEM_SHARED((NPAD, H), jnp.float32),
        pltpu.SemaphoreType.DMA,
        pltpu.SemaphoreType.DMA,
        pltpu.SemaphoreType.DMA,
        pltpu.SemaphoreType.DMA,
    ],
)


ROW_BLK = 2048


def _dense1_body(cnt_p_ref, cnt_a_ref, xp_ref, s1_ref, w1wpl_ref, w1wpr_ref,
                 w1rwl_ref, w1rwr_ref, b1wp_ref, b1rw_ref, out_ref):
    cnt_p = cnt_p_ref[...]
    ind_p = (cnt_p > 0.0).astype(jnp.float32)
    w1sum = jnp.sum(w1wpl_ref[...], axis=0, keepdims=True)
    hp = ind_p * w1sum + jnp.dot(xp_ref[...], w1wpr_ref[...],
                                 preferred_element_type=jnp.float32)
    out_ref[1] = jnp.maximum(hp + b1wp_ref[...], 0.0)
    cnt_a = jnp.maximum(cnt_a_ref[...], 1.0)
    m_a = (s1_ref[0] + s1_ref[1]) / cnt_a
    c = jnp.sum(w1rwr_ref[...], axis=0, keepdims=True) + b1rw_ref[...]
    ha = jnp.dot(m_a, w1rwl_ref[...], preferred_element_type=jnp.float32) + c
    out_ref[0] = jnp.maximum(ha, 0.0)


def _dense2_body(cnt_p_ref, cnt_a_ref, h_ref, s2_ref, w2wpl_ref, w2wpr_ref,
                 w2rwl_ref, w2rwr_ref, b2wp_ref, b2rw_ref, out_ref):
    cnt_p = jnp.maximum(cnt_p_ref[...], 1.0)
    cnt_a = jnp.maximum(cnt_a_ref[...], 1.0)
    out_ref[1] = (jnp.dot(s2_ref[0] / cnt_p, w2wpl_ref[...],
                          preferred_element_type=jnp.float32)
                  + jnp.dot(h_ref[1], w2wpr_ref[...],
                            preferred_element_type=jnp.float32)
                  + b2wp_ref[...])
    out_ref[0] = (jnp.dot(s2_ref[1] / cnt_a, w2rwl_ref[...],
                          preferred_element_type=jnp.float32)
                  + jnp.dot(h_ref[0], w2rwr_ref[...],
                            preferred_element_type=jnp.float32)
                  + b2rw_ref[...])


def _cls_body(ga_ref, gp_ref, out_ref):
    out_ref[...] = jnp.sum(ga_ref[...] * gp_ref[...], axis=-1)


def _row_spec():
    return pl.BlockSpec((ROW_BLK, H), lambda i: (i, 0))


def _pair_spec():
    return pl.BlockSpec((2, ROW_BLK, H), lambda i: (0, i, 0))


def _full_spec():
    return pl.BlockSpec((H, H), lambda i: (0, 0))


def _bias_spec():
    return pl.BlockSpec((1, H), lambda i: (0, 0))


def _cnt_spec():
    return pl.BlockSpec((ROW_BLK, 1), lambda i: (i, 0))


def kernel(x_paper, x_author, edge_index_writes, edge_index_rev,
           edge_label_index, W1_wp_l, W1_wp_r, W1_rw_l, W1_rw_r, W2_wp_l,
           W2_wp_r, W2_rw_l, W2_rw_r, b1_wp, b1_rw, b2_wp, b2_rw):
    ew0 = edge_index_writes[0].astype(jnp.int32)
    ew1 = edge_index_writes[1].astype(jnp.int32)
    x_pad = jnp.pad(x_paper, ((0, NPAD - N), (0, 0)))

    pad1 = E1 - E
    g1 = jnp.pad(ew1, (0, pad1), constant_values=N).reshape(NW, K1, 128)
    s1 = jnp.pad(ew0, (0, pad1), constant_values=N).reshape(NW, K1, 128)
    s1p, cp_p, ca_p = _seg1(x_pad, g1, s1)
    cnt_p = (cp_p[0] + cp_p[1]).reshape(NPAD, 1)
    cnt_a = (ca_p[0] + ca_p[1]).reshape(NPAD, 1)

    grid = (NPAD // ROW_BLK,)
    hcat = pl.pallas_call(
        _dense1_body,
        grid=grid,
        in_specs=[_cnt_spec(), _cnt_spec(), _row_spec(), _pair_spec(),
                  _full_spec(), _full_spec(), _full_spec(), _full_spec(),
                  _bias_spec(), _bias_spec()],
        out_specs=_pair_spec(),
        out_shape=jax.ShapeDtypeStruct((2, NPAD, H), jnp.float32),
    )(cnt_p, cnt_a, x_pad, s1p, W1_wp_l, W1_wp_r, W1_rw_l, W1_rw_r,
      b1_wp[None, :], b1_rw[None, :])

    pad2 = E2 - E
    g2 = jnp.stack([
        jnp.pad(ew0, (0, pad2), constant_values=N),
        jnp.pad(ew1, (0, pad2), constant_values=N) + NPAD,
    ]).reshape(NC, NS, K2, 128)
    s2 = jnp.stack([
        jnp.pad(ew1, (0, pad2), constant_values=N),
        jnp.pad(ew0, (0, pad2), constant_values=N),
    ]).reshape(NC, NS, K2, 128)
    (s2out,) = _seg2(hcat.reshape(NC * NPAD, H), g2, s2)

    hcat2 = pl.pallas_call(
        _dense2_body,
        grid=grid,
        in_specs=[_cnt_spec(), _cnt_spec(), _pair_spec(), _pair_spec(),
                  _full_spec(), _full_spec(), _full_spec(), _full_spec(),
                  _bias_spec(), _bias_spec()],
        out_specs=_pair_spec(),
        out_shape=jax.ShapeDtypeStruct((2, NPAD, H), jnp.float32),
    )(cnt_p, cnt_a, hcat, s2out, W2_wp_l, W2_wp_r, W2_rw_l, W2_rw_r,
      b2_wp[None, :], b2_rw[None, :])

    el0 = edge_label_index[0].astype(jnp.int32)
    el1 = edge_label_index[1].astype(jnp.int32)
    padc = ELP - EL
    ia = jnp.pad(el0, (0, padc), constant_values=N).reshape(NW, KC, 128)
    ib = (jnp.pad(el1, (0, padc), constant_values=N) + NPAD
          ).reshape(NW, KC, 128)
    (part,) = _cls_sc(hcat2.reshape(NC * NPAD, H), ia, ib)
    v = part.reshape(ELP * 16 // H, H)
    FB = 1664
    y = pl.pallas_call(
        _lanefold_body,
        grid=(v.shape[0] // FB,),
        in_specs=[pl.BlockSpec((FB, H), lambda i: (i, 0))],
        out_specs=pl.BlockSpec((FB, H), lambda i: (i, 0)),
        out_shape=jax.ShapeDtypeStruct((v.shape[0], H), jnp.float32),
    )(v)
    return y[:, :8].reshape(ELP)[:EL]


# speedup vs baseline: 3.3161x; 3.0408x over previous
import functools

import jax
import jax.numpy as jnp
from jax import lax
from jax.experimental import pallas as pl
from jax.experimental.pallas import tpu as pltpu
from jax.experimental.pallas import tpu_sc as plsc

N = 10000
H = 128
E = 320000
EL = 50000

NPAD = 10240
NC, NS = 2, 16
NW = NC * NS
SK = 40
K1 = 80
E1 = NW * K1 * 128
K2 = 160
E2 = NS * K2 * 128
ROWB = NPAD // NS

_mesh = plsc.VectorSubcoreMesh(core_axis_name="c", subcore_axis_name="s")


def _zero_buf(buf):
    def zb(t, c):
        buf[t >> 3, pl.ds((t & 7) * 16, 16)] = jnp.zeros((16,), jnp.float32)
        return c
    lax.fori_loop(0, 1024, zb, 0)


def _run_pipeline(tab_hbm, ghw, shw, gidx, sidx, buf0, buf1,
                  gsem0, gsem1, ssem0, ssem1, cntsem,
                  acc, nstages, counts=None):
    def stage(s, c):
        pltpu.sync_copy(ghw.at[pl.ds(s * SK, SK)], gidx)
        pltpu.sync_copy(shw.at[pl.ds(s * SK, SK)], sidx)
        pltpu.async_copy(tab_hbm.at[gidx.at[0]], buf0, gsem0)
        pltpu.async_copy(tab_hbm.at[gidx.at[1]], buf1, gsem1)

        def step(jj, buf, sem):
            pltpu.make_async_copy(tab_hbm.at[gidx.at[jj]], buf, sem).wait()
            pltpu.sync_copy(buf, acc.at[sidx.at[jj]], add=True)
            if counts is not None:
                ones_buf, cpacc, caacc = counts
                pltpu.sync_copy(ones_buf, cpacc.at[gidx.at[jj]], add=True)
                pltpu.sync_copy(ones_buf, caacc.at[sidx.at[jj]], add=True)

            @pl.when(jj + 2 < SK)
            def _():
                pltpu.async_copy(tab_hbm.at[gidx.at[jj + 2]], buf, sem)

        def pair(t, c2):
            step(2 * t, buf0, gsem0)
            step(2 * t + 1, buf1, gsem1)
            return c2
        return lax.fori_loop(0, SK // 2, pair, c)

    lax.fori_loop(0, nstages, stage, 0)


def _seg1_body(x_hbm, g_hbm, s_hbm, out_s, out_cp, out_ca,
               gidx, sidx, buf0, buf1, ones_buf, acc, cpacc, caacc,
               gsem0, gsem1, ssem0, ssem1, cntsem):
    cid = lax.axis_index("c")
    sid = lax.axis_index("s")
    wid = cid * NS + sid
    _zero_buf(buf0)
    for v in range(8):
        ones_buf[pl.ds(v * 16, 16)] = jnp.ones((16,), jnp.float32)
    for k in range(ROWB // 128):
        pltpu.sync_copy(buf0, acc.at[pl.ds(sid * ROWB + k * 128, 128)])
        pltpu.sync_copy(buf0.at[0], cpacc.at[pl.ds(sid * ROWB + k * 128, 128)])
        pltpu.sync_copy(buf0.at[0], caacc.at[pl.ds(sid * ROWB + k * 128, 128)])
    plsc.subcore_barrier()

    _run_pipeline(x_hbm, g_hbm.at[wid], s_hbm.at[wid], gidx, sidx,
                  buf0, buf1, gsem0, gsem1, ssem0, ssem1, cntsem,
                  acc, K1 // SK, counts=(ones_buf, cpacc, caacc))

    plsc.subcore_barrier()
    pltpu.sync_copy(acc.at[pl.ds(sid * ROWB, ROWB)],
                    out_s.at[cid, pl.ds(sid * ROWB, ROWB)])
    pltpu.sync_copy(cpacc.at[pl.ds(sid * ROWB, ROWB)],
                    out_cp.at[cid, pl.ds(sid * ROWB, ROWB)])
    pltpu.sync_copy(caacc.at[pl.ds(sid * ROWB, ROWB)],
                    out_ca.at[cid, pl.ds(sid * ROWB, ROWB)])


def _seg2_body(tab_hbm, g_hbm, s_hbm, out_s, gidx, sidx, buf0, buf1, acc,
               gsem0, gsem1, ssem0, ssem1):
    cid = lax.axis_index("c")
    sid = lax.axis_index("s")
    _zero_buf(buf0)
    for k in range(ROWB // 128):
        pltpu.sync_copy(buf0, acc.at[pl.ds(sid * ROWB + k * 128, 128)])
    plsc.subcore_barrier()

    _run_pipeline(tab_hbm, g_hbm.at[cid, sid], s_hbm.at[cid, sid], gidx, sidx,
                  buf0, buf1, gsem0, gsem1, ssem0, ssem1, None,
                  acc, K2 // SK)

    plsc.subcore_barrier()
    pltpu.sync_copy(acc.at[pl.ds(sid * ROWB, ROWB)],
                    out_s.at[cid, pl.ds(sid * ROWB, ROWB)])


_seg1 = pl.kernel(
    _seg1_body,
    out_type=[jax.ShapeDtypeStruct((NC, NPAD, H), jnp.float32),
              jax.ShapeDtypeStruct((NC, NPAD), jnp.float32),
              jax.ShapeDtypeStruct((NC, NPAD), jnp.float32)],
    mesh=_mesh,
    scratch_types=[
        pltpu.VMEM((SK, 128), jnp.int32),
        pltpu.VMEM((SK, 128), jnp.int32),
        pltpu.VMEM((128, H), jnp.float32),
        pltpu.VMEM((128, H), jnp.float32),
        pltpu.VMEM((128,), jnp.float32),
        pltpu.VMEM_SHARED((NPAD, H), jnp.float32),
        pltpu.VMEM_SHARED((NPAD,), jnp.float32),
        pltpu.VMEM_SHARED((NPAD,), jnp.float32),
        pltpu.SemaphoreType.DMA,
        pltpu.SemaphoreType.DMA,
        pltpu.SemaphoreType.DMA,
        pltpu.SemaphoreType.DMA,
        pltpu.SemaphoreType.DMA,
    ],
)

KC = 13
ELP = NW * KC * 128


def _cls_sc_body(tab_hbm, ia_hbm, ib_hbm, out_hbm, iav, ibv,
                 bufA0, bufB0, bufA1, bufB1, outv,
                 semA0, semB0, semA1, semB1):
    cid = lax.axis_index("c")
    sid = lax.axis_index("s")
    wid = cid * NS + sid
    pltpu.sync_copy(ia_hbm.at[wid], iav)
    pltpu.sync_copy(ib_hbm.at[wid], ibv)

    def issue(j, bufA, bufB, semA, semB):
        pltpu.async_copy(tab_hbm.at[iav.at[j]], bufA, semA)
        pltpu.async_copy(tab_hbm.at[ibv.at[j]], bufB, semB)

    def wait(j, bufA, bufB, semA, semB):
        pltpu.make_async_copy(tab_hbm.at[iav.at[j]], bufA, semA).wait()
        pltpu.make_async_copy(tab_hbm.at[ibv.at[j]], bufB, semB).wait()

    def compute(j, bufA, bufB):
        def row(r, c):
            a = bufA[r, pl.ds(0, 16)] * bufB[r, pl.ds(0, 16)]
            for v in range(1, 8):
                a = a + (bufA[r, pl.ds(v * 16, 16)]
                         * bufB[r, pl.ds(v * 16, 16)])
            outv[j, pl.ds(r * 16, 16)] = a
            return c
        lax.fori_loop(0, 128, row, 0)

    issue(0, bufA0, bufB0, semA0, semB0)

    def pair(t, c):
        j0 = 2 * t
        j1 = 2 * t + 1
        wait(j0, bufA0, bufB0, semA0, semB0)
        issue(j0 + 1, bufA1, bufB1, semA1, semB1)
        compute(j0, bufA0, bufB0)
        wait(j1, bufA1, bufB1, semA1, semB1)

        @pl.when(j1 + 1 < KC)
        def _():
            issue(j1 + 1, bufA0, bufB0, semA0, semB0)
        compute(j1, bufA1, bufB1)
        return c
    lax.fori_loop(0, KC // 2, pair, 0)
    wait(KC - 1, bufA0, bufB0, semA0, semB0)
    compute(KC - 1, bufA0, bufB0)
    pltpu.sync_copy(outv, out_hbm.at[wid])


_cls_sc = pl.kernel(
    _cls_sc_body,
    out_type=[jax.ShapeDtypeStruct((NW, KC, 2048), jnp.float32)],
    mesh=_mesh,
    scratch_types=[
        pltpu.VMEM((KC, 128), jnp.int32),
        pltpu.VMEM((KC, 128), jnp.int32),
        pltpu.VMEM((128, H), jnp.float32),
        pltpu.VMEM((128, H), jnp.float32),
        pltpu.VMEM((128, H), jnp.float32),
        pltpu.VMEM((128, H), jnp.float32),
        pltpu.VMEM((KC, 2048), jnp.float32),
        pltpu.SemaphoreType.DMA,
        pltpu.SemaphoreType.DMA,
        pltpu.SemaphoreType.DMA,
        pltpu.SemaphoreType.DMA,
    ],
)


def _lanefold_body(x_ref, out_ref):
    i = lax.broadcasted_iota(jnp.int32, (H, H), 0)
    j = lax.broadcasted_iota(jnp.int32, (H, H), 1)
    m = ((i // 16) == j).astype(jnp.float32)
    out_ref[...] = jnp.dot(x_ref[...], m, preferred_element_type=jnp.float32)


_seg2 = pl.kernel(
    _seg2_body,
    out_type=[jax.ShapeDtypeStruct((NC, NPAD, H), jnp.float32)],
    mesh=_mesh,
    scratch_types=[
        pltpu.VMEM((SK, 128), jnp.int32),
        pltpu.VMEM((SK, 128), jnp.int32),
        pltpu.VMEM((128, H), jnp.float32),
        pltpu.VMEM((128, H), jnp.float32),
        pltpu.VMEM_SHARED((NPAD, H), jnp.float32),
        pltpu.SemaphoreType.DMA,
        pltpu.SemaphoreType.DMA,
        pltpu.SemaphoreType.DMA,
        pltpu.SemaphoreType.DMA,
    ],
)


ROW_BLK = 2048


def _dense1_body(cnt_p_ref, cnt_a_ref, xp_ref, s1_ref, w1wpl_ref, w1wpr_ref,
                 w1rwl_ref, w1rwr_ref, b1wp_ref, b1rw_ref, out_ref):
    cnt_p = cnt_p_ref[...]
    ind_p = (cnt_p > 0.0).astype(jnp.float32)
    w1sum = jnp.sum(w1wpl_ref[...], axis=0, keepdims=True)
    hp = ind_p * w1sum + jnp.dot(xp_ref[...], w1wpr_ref[...],
                                 preferred_element_type=jnp.float32)
    out_ref[1] = jnp.maximum(hp + b1wp_ref[...], 0.0)
    cnt_a = jnp.maximum(cnt_a_ref[...], 1.0)
    m_a = (s1_ref[0] + s1_ref[1]) / cnt_a
    c = jnp.sum(w1rwr_ref[...], axis=0, keepdims=True) + b1rw_ref[...]
    ha = jnp.dot(m_a, w1rwl_ref[...], preferred_element_type=jnp.float32) + c
    out_ref[0] = jnp.maximum(ha, 0.0)


def _dense2_body(cnt_p_ref, cnt_a_ref, h_ref, s2_ref, w2wpl_ref, w2wpr_ref,
                 w2rwl_ref, w2rwr_ref, b2wp_ref, b2rw_ref, out_ref):
    cnt_p = jnp.maximum(cnt_p_ref[...], 1.0)
    cnt_a = jnp.maximum(cnt_a_ref[...], 1.0)
    out_ref[1] = (jnp.dot(s2_ref[0] / cnt_p, w2wpl_ref[...],
                          preferred_element_type=jnp.float32)
                  + jnp.dot(h_ref[1], w2wpr_ref[...],
                            preferred_element_type=jnp.float32)
                  + b2wp_ref[...])
    out_ref[0] = (jnp.dot(s2_ref[1] / cnt_a, w2rwl_ref[...],
                          preferred_element_type=jnp.float32)
                  + jnp.dot(h_ref[0], w2rwr_ref[...],
                            preferred_element_type=jnp.float32)
                  + b2rw_ref[...])


def _cls_body(ga_ref, gp_ref, out_ref):
    out_ref[...] = jnp.sum(ga_ref[...] * gp_ref[...], axis=-1)


def _row_spec():
    return pl.BlockSpec((ROW_BLK, H), lambda i: (i, 0))


def _pair_spec():
    return pl.BlockSpec((2, ROW_BLK, H), lambda i: (0, i, 0))


def _full_spec():
    return pl.BlockSpec((H, H), lambda i: (0, 0))


def _bias_spec():
    return pl.BlockSpec((1, H), lambda i: (0, 0))


def _cnt_spec():
    return pl.BlockSpec((ROW_BLK, 1), lambda i: (i, 0))


def _pad_idx(arr, target):
    pad = target - arr.shape[0]
    fill = N + (jnp.arange(pad, dtype=jnp.int32) % (NPAD - N))
    return jnp.concatenate([arr, fill])


def kernel(x_paper, x_author, edge_index_writes, edge_index_rev,
           edge_label_index, W1_wp_l, W1_wp_r, W1_rw_l, W1_rw_r, W2_wp_l,
           W2_wp_r, W2_rw_l, W2_rw_r, b1_wp, b1_rw, b2_wp, b2_rw):
    ew0 = edge_index_writes[0].astype(jnp.int32)
    ew1 = edge_index_writes[1].astype(jnp.int32)
    x_pad = jnp.pad(x_paper, ((0, NPAD - N), (0, 0)))

    g1 = _pad_idx(ew1, E1).reshape(NW, K1, 128)
    s1 = _pad_idx(ew0, E1).reshape(NW, K1, 128)
    s1p, cp_p, ca_p = _seg1(x_pad, g1, s1)
    cnt_p = (cp_p[0] + cp_p[1]).reshape(NPAD, 1)
    cnt_a = (ca_p[0] + ca_p[1]).reshape(NPAD, 1)

    grid = (NPAD // ROW_BLK,)
    hcat = pl.pallas_call(
        _dense1_body,
        grid=grid,
        in_specs=[_cnt_spec(), _cnt_spec(), _row_spec(), _pair_spec(),
                  _full_spec(), _full_spec(), _full_spec(), _full_spec(),
                  _bias_spec(), _bias_spec()],
        out_specs=_pair_spec(),
        out_shape=jax.ShapeDtypeStruct((2, NPAD, H), jnp.float32),
    )(cnt_p, cnt_a, x_pad, s1p, W1_wp_l, W1_wp_r, W1_rw_l, W1_rw_r,
      b1_wp[None, :], b1_rw[None, :])

    g2 = jnp.stack([
        _pad_idx(ew0, E2),
        _pad_idx(ew1, E2) + NPAD,
    ]).reshape(NC, NS, K2, 128)
    s2 = jnp.stack([
        _pad_idx(ew1, E2),
        _pad_idx(ew0, E2),
    ]).reshape(NC, NS, K2, 128)
    (s2out,) = _seg2(hcat.reshape(NC * NPAD, H), g2, s2)

    hcat2 = pl.pallas_call(
        _dense2_body,
        grid=grid,
        in_specs=[_cnt_spec(), _cnt_spec(), _pair_spec(), _pair_spec(),
                  _full_spec(), _full_spec(), _full_spec(), _full_spec(),
                  _bias_spec(), _bias_spec()],
        out_specs=_pair_spec(),
        out_shape=jax.ShapeDtypeStruct((2, NPAD, H), jnp.float32),
    )(cnt_p, cnt_a, hcat, s2out, W2_wp_l, W2_wp_r, W2_rw_l, W2_rw_r,
      b2_wp[None, :], b2_rw[None, :])

    el0 = edge_label_index[0].astype(jnp.int32)
    el1 = edge_label_index[1].astype(jnp.int32)
    ia = _pad_idx(el0, ELP).reshape(NW, KC, 128)
    ib = (_pad_idx(el1, ELP) + NPAD).reshape(NW, KC, 128)
    (part,) = _cls_sc(hcat2.reshape(NC * NPAD, H), ia, ib)
    v = part.reshape(ELP * 16 // H, H)
    FB = 1664
    y = pl.pallas_call(
        _lanefold_body,
        grid=(v.shape[0] // FB,),
        in_specs=[pl.BlockSpec((FB, H), lambda i: (i, 0))],
        out_specs=pl.BlockSpec((FB, H), lambda i: (i, 0)),
        out_shape=jax.ShapeDtypeStruct((v.shape[0], H), jnp.float32),
    )(v)
    return y[:, :8].reshape(ELP)[:EL]


# speedup vs baseline: 3.3698x; 1.0162x over previous
import functools

import jax
import jax.numpy as jnp
from jax import lax
from jax.experimental import pallas as pl
from jax.experimental.pallas import tpu as pltpu
from jax.experimental.pallas import tpu_sc as plsc

N = 10000
H = 128
E = 320000
EL = 50000

NPAD = 10240
NC, NS = 2, 16
NW = NC * NS
SK = 40
K1 = 80
E1 = NW * K1 * 128
K2 = 160
E2 = NS * K2 * 128
ROWB = NPAD // NS

_mesh = plsc.VectorSubcoreMesh(core_axis_name="c", subcore_axis_name="s")


def _zero_buf(buf):
    def zb(t, c):
        buf[t >> 3, pl.ds((t & 7) * 16, 16)] = jnp.zeros((16,), jnp.float32)
        return c
    lax.fori_loop(0, 1024, zb, 0)


def _run_pipeline(tab_hbm, ghw, shw, gidx, sidx, buf0, buf1,
                  gsem0, gsem1, ssem0, ssem1, cntsem,
                  acc, nstages, counts=None):
    def stage(s, c):
        pltpu.sync_copy(ghw.at[pl.ds(s * SK, SK)], gidx)
        pltpu.sync_copy(shw.at[pl.ds(s * SK, SK)], sidx)
        pltpu.async_copy(tab_hbm.at[gidx.at[0]], buf0, gsem0)
        pltpu.async_copy(tab_hbm.at[gidx.at[1]], buf1, gsem1)

        def step(jj, buf, sem):
            pltpu.make_async_copy(tab_hbm.at[gidx.at[jj]], buf, sem).wait()
            pltpu.sync_copy(buf, acc.at[sidx.at[jj]], add=True)
            if counts is not None:
                ones_buf, cpacc, caacc = counts
                pltpu.sync_copy(ones_buf, cpacc.at[gidx.at[jj]], add=True)
                pltpu.sync_copy(ones_buf, caacc.at[sidx.at[jj]], add=True)

            @pl.when(jj + 2 < SK)
            def _():
                pltpu.async_copy(tab_hbm.at[gidx.at[jj + 2]], buf, sem)

        def pair(t, c2):
            step(2 * t, buf0, gsem0)
            step(2 * t + 1, buf1, gsem1)
            return c2
        return lax.fori_loop(0, SK // 2, pair, c)

    lax.fori_loop(0, nstages, stage, 0)


def _seg1_body(x_hbm, g_hbm, s_hbm, out_s, out_cp, out_ca,
               gidx, sidx, buf0, buf1, ones_buf, acc, cpacc, caacc,
               gsem0, gsem1, ssem0, ssem1, cntsem):
    cid = lax.axis_index("c")
    sid = lax.axis_index("s")
    wid = cid * NS + sid
    _zero_buf(buf0)
    for v in range(8):
        ones_buf[pl.ds(v * 16, 16)] = jnp.ones((16,), jnp.float32)
    for k in range(ROWB // 128):
        pltpu.sync_copy(buf0, acc.at[pl.ds(sid * ROWB + k * 128, 128)])
        pltpu.sync_copy(buf0.at[0], cpacc.at[pl.ds(sid * ROWB + k * 128, 128)])
        pltpu.sync_copy(buf0.at[0], caacc.at[pl.ds(sid * ROWB + k * 128, 128)])
    plsc.subcore_barrier()

    _run_pipeline(x_hbm, g_hbm.at[wid], s_hbm.at[wid], gidx, sidx,
                  buf0, buf1, gsem0, gsem1, ssem0, ssem1, cntsem,
                  acc, K1 // SK, counts=(ones_buf, cpacc, caacc))

    plsc.subcore_barrier()
    pltpu.sync_copy(acc.at[pl.ds(sid * ROWB, ROWB)],
                    out_s.at[cid, pl.ds(sid * ROWB, ROWB)])
    pltpu.sync_copy(cpacc.at[pl.ds(sid * ROWB, ROWB)],
                    out_cp.at[cid, pl.ds(sid * ROWB, ROWB)])
    pltpu.sync_copy(caacc.at[pl.ds(sid * ROWB, ROWB)],
                    out_ca.at[cid, pl.ds(sid * ROWB, ROWB)])


def _seg2_body(tab_hbm, g_hbm, s_hbm, out_s, gidx, sidx, buf0, buf1, acc,
               gsem0, gsem1, ssem0, ssem1):
    cid = lax.axis_index("c")
    sid = lax.axis_index("s")
    _zero_buf(buf0)
    for k in range(ROWB // 128):
        pltpu.sync_copy(buf0, acc.at[pl.ds(sid * ROWB + k * 128, 128)])
    plsc.subcore_barrier()

    _run_pipeline(tab_hbm.at[cid], g_hbm.at[cid, sid], s_hbm.at[cid, sid],
                  gidx, sidx, buf0, buf1, gsem0, gsem1, ssem0, ssem1, None,
                  acc, K2 // SK)

    plsc.subcore_barrier()
    pltpu.sync_copy(acc.at[pl.ds(sid * ROWB, ROWB)],
                    out_s.at[cid, pl.ds(sid * ROWB, ROWB)])


_seg1 = pl.kernel(
    _seg1_body,
    out_type=[jax.ShapeDtypeStruct((NC, NPAD, H), jnp.float32),
              jax.ShapeDtypeStruct((NC, NPAD), jnp.float32),
              jax.ShapeDtypeStruct((NC, NPAD), jnp.float32)],
    mesh=_mesh,
    scratch_types=[
        pltpu.VMEM((SK, 128), jnp.int32),
        pltpu.VMEM((SK, 128), jnp.int32),
        pltpu.VMEM((128, H), jnp.float32),
        pltpu.VMEM((128, H), jnp.float32),
        pltpu.VMEM((128,), jnp.float32),
        pltpu.VMEM_SHARED((NPAD, H), jnp.float32),
        pltpu.VMEM_SHARED((NPAD,), jnp.float32),
        pltpu.VMEM_SHARED((NPAD,), jnp.float32),
        pltpu.SemaphoreType.DMA,
        pltpu.SemaphoreType.DMA,
        pltpu.SemaphoreType.DMA,
        pltpu.SemaphoreType.DMA,
        pltpu.SemaphoreType.DMA,
    ],
)

KC = 13
ELP = NW * KC * 128


def _cls_sc_body(tab_hbm, ia_hbm, ib_hbm, out_hbm, iav, ibv,
                 bufA0, bufB0, bufA1, bufB1, outv,
                 semA0, semB0, semA1, semB1):
    cid = lax.axis_index("c")
    sid = lax.axis_index("s")
    wid = cid * NS + sid
    tabA = tab_hbm.at[0]
    tabB = tab_hbm.at[1]
    pltpu.sync_copy(ia_hbm.at[wid], iav)
    pltpu.sync_copy(ib_hbm.at[wid], ibv)

    def issue(j, bufA, bufB, semA, semB):
        pltpu.async_copy(tabA.at[iav.at[j]], bufA, semA)
        pltpu.async_copy(tabB.at[ibv.at[j]], bufB, semB)

    def wait(j, bufA, bufB, semA, semB):
        pltpu.make_async_copy(tabA.at[iav.at[j]], bufA, semA).wait()
        pltpu.make_async_copy(tabB.at[ibv.at[j]], bufB, semB).wait()

    def compute(j, bufA, bufB):
        def row(r, c):
            a = bufA[r, pl.ds(0, 16)] * bufB[r, pl.ds(0, 16)]
            for v in range(1, 8):
                a = a + (bufA[r, pl.ds(v * 16, 16)]
                         * bufB[r, pl.ds(v * 16, 16)])
            outv[16 * j + (r >> 3), pl.ds((r & 7) * 16, 16)] = a
            return c
        lax.fori_loop(0, 128, row, 0)

    issue(0, bufA0, bufB0, semA0, semB0)

    def pair(t, c):
        j0 = 2 * t
        j1 = 2 * t + 1
        wait(j0, bufA0, bufB0, semA0, semB0)
        issue(j0 + 1, bufA1, bufB1, semA1, semB1)
        compute(j0, bufA0, bufB0)
        wait(j1, bufA1, bufB1, semA1, semB1)

        @pl.when(j1 + 1 < KC)
        def _():
            issue(j1 + 1, bufA0, bufB0, semA0, semB0)
        compute(j1, bufA1, bufB1)
        return c
    lax.fori_loop(0, KC // 2, pair, 0)
    wait(KC - 1, bufA0, bufB0, semA0, semB0)
    compute(KC - 1, bufA0, bufB0)
    pltpu.sync_copy(outv, out_hbm.at[pl.ds(wid * 16 * KC, 16 * KC)])


_cls_sc = pl.kernel(
    _cls_sc_body,
    out_type=[jax.ShapeDtypeStruct((NW * KC * 16, H), jnp.float32)],
    mesh=_mesh,
    scratch_types=[
        pltpu.VMEM((KC, 128), jnp.int32),
        pltpu.VMEM((KC, 128), jnp.int32),
        pltpu.VMEM((128, H), jnp.float32),
        pltpu.VMEM((128, H), jnp.float32),
        pltpu.VMEM((128, H), jnp.float32),
        pltpu.VMEM((128, H), jnp.float32),
        pltpu.VMEM((KC * 16, H), jnp.float32),
        pltpu.SemaphoreType.DMA,
        pltpu.SemaphoreType.DMA,
        pltpu.SemaphoreType.DMA,
        pltpu.SemaphoreType.DMA,
    ],
)


def _lanefold_body(x_ref, out_ref):
    i = lax.broadcasted_iota(jnp.int32, (H, H), 0)
    j = lax.broadcasted_iota(jnp.int32, (H, H), 1)
    m = ((i // 16) == j).astype(jnp.float32)
    out_ref[...] = jnp.dot(x_ref[...], m, preferred_element_type=jnp.float32)


_seg2 = pl.kernel(
    _seg2_body,
    out_type=[jax.ShapeDtypeStruct((NC, NPAD, H), jnp.float32)],
    mesh=_mesh,
    scratch_types=[
        pltpu.VMEM((SK, 128), jnp.int32),
        pltpu.VMEM((SK, 128), jnp.int32),
        pltpu.VMEM((128, H), jnp.float32),
        pltpu.VMEM((128, H), jnp.float32),
        pltpu.VMEM_SHARED((NPAD, H), jnp.float32),
        pltpu.SemaphoreType.DMA,
        pltpu.SemaphoreType.DMA,
        pltpu.SemaphoreType.DMA,
        pltpu.SemaphoreType.DMA,
    ],
)


ROW_BLK = 2048


def _dense1_body(cnt_p_ref, cnt_a_ref, xp_ref, s1_ref, w1wpl_ref, w1wpr_ref,
                 w1rwl_ref, w1rwr_ref, b1wp_ref, b1rw_ref, out_ref):
    cnt_p = cnt_p_ref[...]
    ind_p = (cnt_p > 0.0).astype(jnp.float32)
    w1sum = jnp.sum(w1wpl_ref[...], axis=0, keepdims=True)
    hp = ind_p * w1sum + jnp.dot(xp_ref[...], w1wpr_ref[...],
                                 preferred_element_type=jnp.float32)
    out_ref[1] = jnp.maximum(hp + b1wp_ref[...], 0.0)
    cnt_a = jnp.maximum(cnt_a_ref[...], 1.0)
    m_a = (s1_ref[0] + s1_ref[1]) / cnt_a
    c = jnp.sum(w1rwr_ref[...], axis=0, keepdims=True) + b1rw_ref[...]
    ha = jnp.dot(m_a, w1rwl_ref[...], preferred_element_type=jnp.float32) + c
    out_ref[0] = jnp.maximum(ha, 0.0)


def _dense2_body(cnt_p_ref, cnt_a_ref, h_ref, s2_ref, w2wpl_ref, w2wpr_ref,
                 w2rwl_ref, w2rwr_ref, b2wp_ref, b2rw_ref, out_ref):
    cnt_p = jnp.maximum(cnt_p_ref[...], 1.0)
    cnt_a = jnp.maximum(cnt_a_ref[...], 1.0)
    out_ref[1] = (jnp.dot(s2_ref[0] / cnt_p, w2wpl_ref[...],
                          preferred_element_type=jnp.float32)
                  + jnp.dot(h_ref[1], w2wpr_ref[...],
                            preferred_element_type=jnp.float32)
                  + b2wp_ref[...])
    out_ref[0] = (jnp.dot(s2_ref[1] / cnt_a, w2rwl_ref[...],
                          preferred_element_type=jnp.float32)
                  + jnp.dot(h_ref[0], w2rwr_ref[...],
                            preferred_element_type=jnp.float32)
                  + b2rw_ref[...])


def _cls_body(ga_ref, gp_ref, out_ref):
    out_ref[...] = jnp.sum(ga_ref[...] * gp_ref[...], axis=-1)


def _row_spec():
    return pl.BlockSpec((ROW_BLK, H), lambda i: (i, 0))


def _pair_spec():
    return pl.BlockSpec((2, ROW_BLK, H), lambda i: (0, i, 0))


def _full_spec():
    return pl.BlockSpec((H, H), lambda i: (0, 0))


def _bias_spec():
    return pl.BlockSpec((1, H), lambda i: (0, 0))


def _cnt_spec():
    return pl.BlockSpec((ROW_BLK, 1), lambda i: (i, 0))


def _pad_idx(arr, target):
    pad = target - arr.shape[0]
    fill = N + (jnp.arange(pad, dtype=jnp.int32) % (NPAD - N))
    return jnp.concatenate([arr, fill])


def kernel(x_paper, x_author, edge_index_writes, edge_index_rev,
           edge_label_index, W1_wp_l, W1_wp_r, W1_rw_l, W1_rw_r, W2_wp_l,
           W2_wp_r, W2_rw_l, W2_rw_r, b1_wp, b1_rw, b2_wp, b2_rw):
    ew0 = edge_index_writes[0].astype(jnp.int32)
    ew1 = edge_index_writes[1].astype(jnp.int32)
    x_pad = jnp.pad(x_paper, ((0, NPAD - N), (0, 0)))

    g1 = _pad_idx(ew1, E1).reshape(NW, K1, 128)
    s1 = _pad_idx(ew0, E1).reshape(NW, K1, 128)
    s1p, cp_p, ca_p = _seg1(x_pad, g1, s1)
    cnt_p = (cp_p[0] + cp_p[1]).reshape(NPAD, 1)
    cnt_a = (ca_p[0] + ca_p[1]).reshape(NPAD, 1)

    grid = (NPAD // ROW_BLK,)
    hcat = pl.pallas_call(
        _dense1_body,
        grid=grid,
        in_specs=[_cnt_spec(), _cnt_spec(), _row_spec(), _pair_spec(),
                  _full_spec(), _full_spec(), _full_spec(), _full_spec(),
                  _bias_spec(), _bias_spec()],
        out_specs=_pair_spec(),
        out_shape=jax.ShapeDtypeStruct((2, NPAD, H), jnp.float32),
    )(cnt_p, cnt_a, x_pad, s1p, W1_wp_l, W1_wp_r, W1_rw_l, W1_rw_r,
      b1_wp[None, :], b1_rw[None, :])

    g2 = jnp.stack([
        _pad_idx(ew0, E2),
        _pad_idx(ew1, E2),
    ]).reshape(NC, NS, K2, 128)
    s2 = jnp.stack([
        _pad_idx(ew1, E2),
        _pad_idx(ew0, E2),
    ]).reshape(NC, NS, K2, 128)
    (s2out,) = _seg2(hcat, g2, s2)

    hcat2 = pl.pallas_call(
        _dense2_body,
        grid=grid,
        in_specs=[_cnt_spec(), _cnt_spec(), _pair_spec(), _pair_spec(),
                  _full_spec(), _full_spec(), _full_spec(), _full_spec(),
                  _bias_spec(), _bias_spec()],
        out_specs=_pair_spec(),
        out_shape=jax.ShapeDtypeStruct((2, NPAD, H), jnp.float32),
    )(cnt_p, cnt_a, hcat, s2out, W2_wp_l, W2_wp_r, W2_rw_l, W2_rw_r,
      b2_wp[None, :], b2_rw[None, :])

    el0 = edge_label_index[0].astype(jnp.int32)
    el1 = edge_label_index[1].astype(jnp.int32)
    ia = _pad_idx(el0, ELP).reshape(NW, KC, 128)
    ib = _pad_idx(el1, ELP).reshape(NW, KC, 128)
    (v,) = _cls_sc(hcat2, ia, ib)
    FB = 1664
    y = pl.pallas_call(
        _lanefold_body,
        grid=(v.shape[0] // FB,),
        in_specs=[pl.BlockSpec((FB, H), lambda i: (i, 0))],
        out_specs=pl.BlockSpec((FB, H), lambda i: (i, 0)),
        out_shape=jax.ShapeDtypeStruct((v.shape[0], H), jnp.float32),
    )(v)
    return y[:, :8].reshape(ELP)[:EL]


# speedup vs baseline: 3.4043x; 1.0102x over previous
import functools

import jax
import jax.numpy as jnp
from jax import lax
from jax.experimental import pallas as pl
from jax.experimental.pallas import tpu as pltpu
from jax.experimental.pallas import tpu_sc as plsc

N = 10000
H = 128
E = 320000
EL = 50000

NPAD = 10240
NC, NS = 2, 16
NW = NC * NS
SK = 40
K1 = 80
E1 = NW * K1 * 128
K2 = 160
E2 = NS * K2 * 128
ROWB = NPAD // NS

_mesh = plsc.VectorSubcoreMesh(core_axis_name="c", subcore_axis_name="s")


def _zero_buf(buf):
    def zb(t, c):
        buf[t >> 3, pl.ds((t & 7) * 16, 16)] = jnp.zeros((16,), jnp.float32)
        return c
    lax.fori_loop(0, 1024, zb, 0)


def _run_pipeline(tab_hbm, ghw, shw, gidx, sidx, buf0, buf1,
                  gsem0, gsem1, ssem0, ssem1, cntsem,
                  acc, nstages, counts=None):
    def stage(s, c):
        pltpu.sync_copy(ghw.at[pl.ds(s * SK, SK)], gidx)
        pltpu.sync_copy(shw.at[pl.ds(s * SK, SK)], sidx)
        pltpu.async_copy(tab_hbm.at[gidx.at[0]], buf0, gsem0)
        pltpu.async_copy(tab_hbm.at[gidx.at[1]], buf1, gsem1)

        def step(jj, buf, sem):
            pltpu.make_async_copy(tab_hbm.at[gidx.at[jj]], buf, sem).wait()
            pltpu.sync_copy(buf, acc.at[sidx.at[jj]], add=True)
            if counts is not None:
                ones_buf, cpacc, caacc = counts
                pltpu.async_copy(ones_buf, cpacc.at[gidx.at[jj]], cntsem,
                                 add=True)
                pltpu.async_copy(ones_buf, caacc.at[sidx.at[jj]], cntsem,
                                 add=True)

                @pl.when(jj >= 2)
                def _():
                    pltpu.make_async_copy(ones_buf, cpacc.at[gidx.at[jj - 2]],
                                          cntsem).wait()
                    pltpu.make_async_copy(ones_buf, caacc.at[sidx.at[jj - 2]],
                                          cntsem).wait()

            @pl.when(jj + 2 < SK)
            def _():
                pltpu.async_copy(tab_hbm.at[gidx.at[jj + 2]], buf, sem)

        def pair(t, c2):
            step(2 * t, buf0, gsem0)
            step(2 * t + 1, buf1, gsem1)
            return c2
        c = lax.fori_loop(0, SK // 2, pair, c)
        if counts is not None:
            ones_buf, cpacc, caacc = counts
            for jj in (SK - 2, SK - 1):
                pltpu.make_async_copy(ones_buf, cpacc.at[gidx.at[jj]],
                                      cntsem).wait()
                pltpu.make_async_copy(ones_buf, caacc.at[sidx.at[jj]],
                                      cntsem).wait()
        return c

    lax.fori_loop(0, nstages, stage, 0)


def _seg1_body(x_hbm, g_hbm, s_hbm, out_s, out_cp, out_ca,
               gidx, sidx, buf0, buf1, ones_buf, acc, cpacc, caacc,
               gsem0, gsem1, ssem0, ssem1, cntsem):
    cid = lax.axis_index("c")
    sid = lax.axis_index("s")
    wid = cid * NS + sid
    _zero_buf(buf0)
    for v in range(8):
        ones_buf[pl.ds(v * 16, 16)] = jnp.ones((16,), jnp.float32)
    for k in range(ROWB // 128):
        pltpu.sync_copy(buf0, acc.at[pl.ds(sid * ROWB + k * 128, 128)])
        pltpu.sync_copy(buf0.at[0], cpacc.at[pl.ds(sid * ROWB + k * 128, 128)])
        pltpu.sync_copy(buf0.at[0], caacc.at[pl.ds(sid * ROWB + k * 128, 128)])
    plsc.subcore_barrier()

    _run_pipeline(x_hbm, g_hbm.at[wid], s_hbm.at[wid], gidx, sidx,
                  buf0, buf1, gsem0, gsem1, ssem0, ssem1, cntsem,
                  acc, K1 // SK, counts=(ones_buf, cpacc, caacc))

    plsc.subcore_barrier()
    pltpu.sync_copy(acc.at[pl.ds(sid * ROWB, ROWB)],
                    out_s.at[cid, pl.ds(sid * ROWB, ROWB)])
    pltpu.sync_copy(cpacc.at[pl.ds(sid * ROWB, ROWB)],
                    out_cp.at[cid, pl.ds(sid * ROWB, ROWB)])
    pltpu.sync_copy(caacc.at[pl.ds(sid * ROWB, ROWB)],
                    out_ca.at[cid, pl.ds(sid * ROWB, ROWB)])


def _seg2_body(tab_hbm, g_hbm, s_hbm, out_s, gidx, sidx, buf0, buf1, acc,
               gsem0, gsem1, ssem0, ssem1):
    cid = lax.axis_index("c")
    sid = lax.axis_index("s")
    _zero_buf(buf0)
    for k in range(ROWB // 128):
        pltpu.sync_copy(buf0, acc.at[pl.ds(sid * ROWB + k * 128, 128)])
    plsc.subcore_barrier()

    _run_pipeline(tab_hbm.at[cid], g_hbm.at[cid, sid], s_hbm.at[cid, sid],
                  gidx, sidx, buf0, buf1, gsem0, gsem1, ssem0, ssem1, None,
                  acc, K2 // SK)

    plsc.subcore_barrier()
    pltpu.sync_copy(acc.at[pl.ds(sid * ROWB, ROWB)],
                    out_s.at[cid, pl.ds(sid * ROWB, ROWB)])


_seg1 = pl.kernel(
    _seg1_body,
    out_type=[jax.ShapeDtypeStruct((NC, NPAD, H), jnp.float32),
              jax.ShapeDtypeStruct((NC, NPAD), jnp.float32),
              jax.ShapeDtypeStruct((NC, NPAD), jnp.float32)],
    mesh=_mesh,
    scratch_types=[
        pltpu.VMEM((SK, 128), jnp.int32),
        pltpu.VMEM((SK, 128), jnp.int32),
        pltpu.VMEM((128, H), jnp.float32),
        pltpu.VMEM((128, H), jnp.float32),
        pltpu.VMEM((128,), jnp.float32),
        pltpu.VMEM_SHARED((NPAD, H), jnp.float32),
        pltpu.VMEM_SHARED((NPAD,), jnp.float32),
        pltpu.VMEM_SHARED((NPAD,), jnp.float32),
        pltpu.SemaphoreType.DMA,
        pltpu.SemaphoreType.DMA,
        pltpu.SemaphoreType.DMA,
        pltpu.SemaphoreType.DMA,
        pltpu.SemaphoreType.DMA,
    ],
)

KC = 13
ELP = NW * KC * 128


def _cls_sc_body(tab_hbm, ia_hbm, ib_hbm, out_hbm, iav, ibv,
                 bufA0, bufB0, bufA1, bufB1, outv,
                 semA0, semB0, semA1, semB1):
    cid = lax.axis_index("c")
    sid = lax.axis_index("s")
    wid = cid * NS + sid
    tabA = tab_hbm.at[0]
    tabB = tab_hbm.at[1]
    pltpu.sync_copy(ia_hbm.at[wid], iav)
    pltpu.sync_copy(ib_hbm.at[wid], ibv)

    def issue(j, bufA, bufB, semA, semB):
        pltpu.async_copy(tabA.at[iav.at[j]], bufA, semA)
        pltpu.async_copy(tabB.at[ibv.at[j]], bufB, semB)

    def wait(j, bufA, bufB, semA, semB):
        pltpu.make_async_copy(tabA.at[iav.at[j]], bufA, semA).wait()
        pltpu.make_async_copy(tabB.at[ibv.at[j]], bufB, semB).wait()

    def compute(j, bufA, bufB):
        def row(r, c):
            a = bufA[r, pl.ds(0, 16)] * bufB[r, pl.ds(0, 16)]
            for v in range(1, 8):
                a = a + (bufA[r, pl.ds(v * 16, 16)]
                         * bufB[r, pl.ds(v * 16, 16)])
            outv[16 * j + (r >> 3), pl.ds((r & 7) * 16, 16)] = a
            return c
        lax.fori_loop(0, 128, row, 0)

    issue(0, bufA0, bufB0, semA0, semB0)

    def pair(t, c):
        j0 = 2 * t
        j1 = 2 * t + 1
        wait(j0, bufA0, bufB0, semA0, semB0)
        issue(j0 + 1, bufA1, bufB1, semA1, semB1)
        compute(j0, bufA0, bufB0)
        wait(j1, bufA1, bufB1, semA1, semB1)

        @pl.when(j1 + 1 < KC)
        def _():
            issue(j1 + 1, bufA0, bufB0, semA0, semB0)
        compute(j1, bufA1, bufB1)
        return c
    lax.fori_loop(0, KC // 2, pair, 0)
    wait(KC - 1, bufA0, bufB0, semA0, semB0)
    compute(KC - 1, bufA0, bufB0)
    pltpu.sync_copy(outv, out_hbm.at[pl.ds(wid * 16 * KC, 16 * KC)])


_cls_sc = pl.kernel(
    _cls_sc_body,
    out_type=[jax.ShapeDtypeStruct((NW * KC * 16, H), jnp.float32)],
    mesh=_mesh,
    scratch_types=[
        pltpu.VMEM((KC, 128), jnp.int32),
        pltpu.VMEM((KC, 128), jnp.int32),
        pltpu.VMEM((128, H), jnp.float32),
        pltpu.VMEM((128, H), jnp.float32),
        pltpu.VMEM((128, H), jnp.float32),
        pltpu.VMEM((128, H), jnp.float32),
        pltpu.VMEM((KC * 16, H), jnp.float32),
        pltpu.SemaphoreType.DMA,
        pltpu.SemaphoreType.DMA,
        pltpu.SemaphoreType.DMA,
        pltpu.SemaphoreType.DMA,
    ],
)


def _lanefold_body(x_ref, out_ref):
    i = lax.broadcasted_iota(jnp.int32, (H, H), 0)
    j = lax.broadcasted_iota(jnp.int32, (H, H), 1)
    m = ((i // 16) == j).astype(jnp.float32)
    out_ref[...] = jnp.dot(x_ref[...], m, preferred_element_type=jnp.float32)


_seg2 = pl.kernel(
    _seg2_body,
    out_type=[jax.ShapeDtypeStruct((NC, NPAD, H), jnp.float32)],
    mesh=_mesh,
    scratch_types=[
        pltpu.VMEM((SK, 128), jnp.int32),
        pltpu.VMEM((SK, 128), jnp.int32),
        pltpu.VMEM((128, H), jnp.float32),
        pltpu.VMEM((128, H), jnp.float32),
        pltpu.VMEM_SHARED((NPAD, H), jnp.float32),
        pltpu.SemaphoreType.DMA,
        pltpu.SemaphoreType.DMA,
        pltpu.SemaphoreType.DMA,
        pltpu.SemaphoreType.DMA,
    ],
)


ROW_BLK = 2048


def _dense1_body(cnt_p_ref, cnt_a_ref, xp_ref, s1_ref, w1wpl_ref, w1wpr_ref,
                 w1rwl_ref, w1rwr_ref, b1wp_ref, b1rw_ref, out_ref):
    cnt_p = cnt_p_ref[...]
    ind_p = (cnt_p > 0.0).astype(jnp.float32)
    w1sum = jnp.sum(w1wpl_ref[...], axis=0, keepdims=True)
    hp = ind_p * w1sum + jnp.dot(xp_ref[...], w1wpr_ref[...],
                                 preferred_element_type=jnp.float32)
    out_ref[1] = jnp.maximum(hp + b1wp_ref[...], 0.0)
    cnt_a = jnp.maximum(cnt_a_ref[...], 1.0)
    m_a = (s1_ref[0] + s1_ref[1]) / cnt_a
    c = jnp.sum(w1rwr_ref[...], axis=0, keepdims=True) + b1rw_ref[...]
    ha = jnp.dot(m_a, w1rwl_ref[...], preferred_element_type=jnp.float32) + c
    out_ref[0] = jnp.maximum(ha, 0.0)


def _dense2_body(cnt_p_ref, cnt_a_ref, h_ref, s2_ref, w2wpl_ref, w2wpr_ref,
                 w2rwl_ref, w2rwr_ref, b2wp_ref, b2rw_ref, out_ref):
    cnt_p = jnp.maximum(cnt_p_ref[...], 1.0)
    cnt_a = jnp.maximum(cnt_a_ref[...], 1.0)
    out_ref[1] = (jnp.dot(s2_ref[0] / cnt_p, w2wpl_ref[...],
                          preferred_element_type=jnp.float32)
                  + jnp.dot(h_ref[1], w2wpr_ref[...],
                            preferred_element_type=jnp.float32)
                  + b2wp_ref[...])
    out_ref[0] = (jnp.dot(s2_ref[1] / cnt_a, w2rwl_ref[...],
                          preferred_element_type=jnp.float32)
                  + jnp.dot(h_ref[0], w2rwr_ref[...],
                            preferred_element_type=jnp.float32)
                  + b2rw_ref[...])


def _cls_body(ga_ref, gp_ref, out_ref):
    out_ref[...] = jnp.sum(ga_ref[...] * gp_ref[...], axis=-1)


def _row_spec():
    return pl.BlockSpec((ROW_BLK, H), lambda i: (i, 0))


def _pair_spec():
    return pl.BlockSpec((2, ROW_BLK, H), lambda i: (0, i, 0))


def _full_spec():
    return pl.BlockSpec((H, H), lambda i: (0, 0))


def _bias_spec():
    return pl.BlockSpec((1, H), lambda i: (0, 0))


def _cnt_spec():
    return pl.BlockSpec((ROW_BLK, 1), lambda i: (i, 0))


def _pad_idx(arr, target):
    pad = target - arr.shape[0]
    fill = N + (jnp.arange(pad, dtype=jnp.int32) % (NPAD - N))
    return jnp.concatenate([arr, fill])


def kernel(x_paper, x_author, edge_index_writes, edge_index_rev,
           edge_label_index, W1_wp_l, W1_wp_r, W1_rw_l, W1_rw_r, W2_wp_l,
           W2_wp_r, W2_rw_l, W2_rw_r, b1_wp, b1_rw, b2_wp, b2_rw):
    ew0 = edge_index_writes[0].astype(jnp.int32)
    ew1 = edge_index_writes[1].astype(jnp.int32)
    x_pad = jnp.pad(x_paper, ((0, NPAD - N), (0, 0)))

    g1 = _pad_idx(ew1, E1).reshape(NW, K1, 128)
    s1 = _pad_idx(ew0, E1).reshape(NW, K1, 128)
    s1p, cp_p, ca_p = _seg1(x_pad, g1, s1)
    cnt_p = (cp_p[0] + cp_p[1]).reshape(NPAD, 1)
    cnt_a = (ca_p[0] + ca_p[1]).reshape(NPAD, 1)

    grid = (NPAD // ROW_BLK,)
    hcat = pl.pallas_call(
        _dense1_body,
        grid=grid,
        in_specs=[_cnt_spec(), _cnt_spec(), _row_spec(), _pair_spec(),
                  _full_spec(), _full_spec(), _full_spec(), _full_spec(),
                  _bias_spec(), _bias_spec()],
        out_specs=_pair_spec(),
        out_shape=jax.ShapeDtypeStruct((2, NPAD, H), jnp.float32),
    )(cnt_p, cnt_a, x_pad, s1p, W1_wp_l, W1_wp_r, W1_rw_l, W1_rw_r,
      b1_wp[None, :], b1_rw[None, :])

    g2 = jnp.stack([
        _pad_idx(ew0, E2),
        _pad_idx(ew1, E2),
    ]).reshape(NC, NS, K2, 128)
    s2 = jnp.stack([
        _pad_idx(ew1, E2),
        _pad_idx(ew0, E2),
    ]).reshape(NC, NS, K2, 128)
    (s2out,) = _seg2(hcat, g2, s2)

    hcat2 = pl.pallas_call(
        _dense2_body,
        grid=grid,
        in_specs=[_cnt_spec(), _cnt_spec(), _pair_spec(), _pair_spec(),
                  _full_spec(), _full_spec(), _full_spec(), _full_spec(),
                  _bias_spec(), _bias_spec()],
        out_specs=_pair_spec(),
        out_shape=jax.ShapeDtypeStruct((2, NPAD, H), jnp.float32),
    )(cnt_p, cnt_a, hcat, s2out, W2_wp_l, W2_wp_r, W2_rw_l, W2_rw_r,
      b2_wp[None, :], b2_rw[None, :])

    el0 = edge_label_index[0].astype(jnp.int32)
    el1 = edge_label_index[1].astype(jnp.int32)
    ia = _pad_idx(el0, ELP).reshape(NW, KC, 128)
    ib = _pad_idx(el1, ELP).reshape(NW, KC, 128)
    (v,) = _cls_sc(hcat2, ia, ib)
    FB = 1664
    y = pl.pallas_call(
        _lanefold_body,
        grid=(v.shape[0] // FB,),
        in_specs=[pl.BlockSpec((FB, H), lambda i: (i, 0))],
        out_specs=pl.BlockSpec((FB, H), lambda i: (i, 0)),
        out_shape=jax.ShapeDtypeStruct((v.shape[0], H), jnp.float32),
    )(v)
    return y[:, :8].reshape(ELP)[:EL]


# speedup vs baseline: 3.4089x; 1.0013x over previous
import jax
import jax.numpy as jnp
from jax import lax
from jax.experimental import pallas as pl
from jax.experimental.pallas import tpu as pltpu
from jax.experimental.pallas import tpu_sc as plsc

N = 10000
H = 128
E = 320000
EL = 50000

NPAD = 10240
NC, NS = 2, 16
NW = NC * NS
SK = 40
K1 = 80
E1 = NW * K1 * 128
K2 = 160
E2 = NS * K2 * 128
ROWB = NPAD // NS

_mesh = plsc.VectorSubcoreMesh(core_axis_name="c", subcore_axis_name="s")


def _zero_buf(buf):
    def zb(t, c):
        buf[t >> 3, pl.ds((t & 7) * 16, 16)] = jnp.zeros((16,), jnp.float32)
        return c
    lax.fori_loop(0, 1024, zb, 0)


def _run_pipeline(tab_hbm, ghw, shw, gidx, sidx, buf0, buf1,
                  gsem0, gsem1, ssem0, ssem1, cntsem,
                  acc, nstages, counts=None):
    def stage(s, c):
        pltpu.sync_copy(ghw.at[pl.ds(s * SK, SK)], gidx)
        pltpu.sync_copy(shw.at[pl.ds(s * SK, SK)], sidx)
        pltpu.async_copy(tab_hbm.at[gidx.at[0]], buf0, gsem0)
        pltpu.async_copy(tab_hbm.at[gidx.at[1]], buf1, gsem1)

        def step(jj, buf, sem):
            pltpu.make_async_copy(tab_hbm.at[gidx.at[jj]], buf, sem).wait()
            pltpu.sync_copy(buf, acc.at[sidx.at[jj]], add=True)
            if counts is not None:
                ones_buf, cpacc, caacc = counts
                pltpu.async_copy(ones_buf, cpacc.at[gidx.at[jj]], cntsem,
                                 add=True)
                pltpu.async_copy(ones_buf, caacc.at[sidx.at[jj]], cntsem,
                                 add=True)

                @pl.when(jj >= 2)
                def _():
                    pltpu.make_async_copy(ones_buf, cpacc.at[gidx.at[jj - 2]],
                                          cntsem).wait()
                    pltpu.make_async_copy(ones_buf, caacc.at[sidx.at[jj - 2]],
                                          cntsem).wait()

            @pl.when(jj + 2 < SK)
            def _():
                pltpu.async_copy(tab_hbm.at[gidx.at[jj + 2]], buf, sem)

        def pair(t, c2):
            step(2 * t, buf0, gsem0)
            step(2 * t + 1, buf1, gsem1)
            return c2
        c = lax.fori_loop(0, SK // 2, pair, c)
        if counts is not None:
            ones_buf, cpacc, caacc = counts
            for jj in (SK - 2, SK - 1):
                pltpu.make_async_copy(ones_buf, cpacc.at[gidx.at[jj]],
                                      cntsem).wait()
                pltpu.make_async_copy(ones_buf, caacc.at[sidx.at[jj]],
                                      cntsem).wait()
        return c

    lax.fori_loop(0, nstages, stage, 0)


def _seg1_body(x_hbm, g_hbm, s_hbm, out_s, out_cp, out_ca,
               gidx, sidx, buf0, buf1, ones_buf, acc, cpacc, caacc,
               gsem0, gsem1, ssem0, ssem1, cntsem):
    cid = lax.axis_index("c")
    sid = lax.axis_index("s")
    wid = cid * NS + sid
    _zero_buf(buf0)
    for v in range(8):
        ones_buf[pl.ds(v * 16, 16)] = jnp.ones((16,), jnp.float32)
    for k in range(ROWB // 128):
        pltpu.sync_copy(buf0, acc.at[pl.ds(sid * ROWB + k * 128, 128)])
        pltpu.sync_copy(buf0.at[0], cpacc.at[pl.ds(sid * ROWB + k * 128, 128)])
        pltpu.sync_copy(buf0.at[0], caacc.at[pl.ds(sid * ROWB + k * 128, 128)])
    plsc.subcore_barrier()

    _run_pipeline(x_hbm, g_hbm.at[wid], s_hbm.at[wid], gidx, sidx,
                  buf0, buf1, gsem0, gsem1, ssem0, ssem1, cntsem,
                  acc, K1 // SK, counts=(ones_buf, cpacc, caacc))

    plsc.subcore_barrier()
    pltpu.sync_copy(acc.at[pl.ds(sid * ROWB, ROWB)],
                    out_s.at[cid, pl.ds(sid * ROWB, ROWB)])
    pltpu.sync_copy(cpacc.at[pl.ds(sid * ROWB, ROWB)],
                    out_cp.at[cid, pl.ds(sid * ROWB, ROWB)])
    pltpu.sync_copy(caacc.at[pl.ds(sid * ROWB, ROWB)],
                    out_ca.at[cid, pl.ds(sid * ROWB, ROWB)])


def _seg2_body(tab_hbm, g_hbm, s_hbm, out_s, gidx, sidx, buf0, buf1, acc,
               gsem0, gsem1, ssem0, ssem1):
    cid = lax.axis_index("c")
    sid = lax.axis_index("s")
    _zero_buf(buf0)
    for k in range(ROWB // 128):
        pltpu.sync_copy(buf0, acc.at[pl.ds(sid * ROWB + k * 128, 128)])
    plsc.subcore_barrier()

    _run_pipeline(tab_hbm.at[cid], g_hbm.at[cid, sid], s_hbm.at[cid, sid],
                  gidx, sidx, buf0, buf1, gsem0, gsem1, ssem0, ssem1, None,
                  acc, K2 // SK)

    plsc.subcore_barrier()
    pltpu.sync_copy(acc.at[pl.ds(sid * ROWB, ROWB)],
                    out_s.at[cid, pl.ds(sid * ROWB, ROWB)])


_seg1 = pl.kernel(
    _seg1_body,
    out_type=[jax.ShapeDtypeStruct((NC, NPAD, H), jnp.float32),
              jax.ShapeDtypeStruct((NC, NPAD), jnp.float32),
              jax.ShapeDtypeStruct((NC, NPAD), jnp.float32)],
    mesh=_mesh,
    scratch_types=[
        pltpu.VMEM((SK, 128), jnp.int32),
        pltpu.VMEM((SK, 128), jnp.int32),
        pltpu.VMEM((128, H), jnp.float32),
        pltpu.VMEM((128, H), jnp.float32),
        pltpu.VMEM((128,), jnp.float32),
        pltpu.VMEM_SHARED((NPAD, H), jnp.float32),
        pltpu.VMEM_SHARED((NPAD,), jnp.float32),
        pltpu.VMEM_SHARED((NPAD,), jnp.float32),
        pltpu.SemaphoreType.DMA,
        pltpu.SemaphoreType.DMA,
        pltpu.SemaphoreType.DMA,
        pltpu.SemaphoreType.DMA,
        pltpu.SemaphoreType.DMA,
    ],
)

KC = 13
ELP = NW * KC * 128


def _cls_sc_body(tab_hbm, ia_hbm, ib_hbm, out_hbm, iav, ibv,
                 bufA0, bufB0, bufA1, bufB1, outv,
                 semA0, semB0, semA1, semB1):
    cid = lax.axis_index("c")
    sid = lax.axis_index("s")
    wid = cid * NS + sid
    tabA = tab_hbm.at[0]
    tabB = tab_hbm.at[1]
    pltpu.sync_copy(ia_hbm.at[wid], iav)
    pltpu.sync_copy(ib_hbm.at[wid], ibv)

    def issue(j, bufA, bufB, semA, semB):
        pltpu.async_copy(tabA.at[iav.at[j]], bufA, semA)
        pltpu.async_copy(tabB.at[ibv.at[j]], bufB, semB)

    def wait(j, bufA, bufB, semA, semB):
        pltpu.make_async_copy(tabA.at[iav.at[j]], bufA, semA).wait()
        pltpu.make_async_copy(tabB.at[ibv.at[j]], bufB, semB).wait()

    def compute(j, bufA, bufB):
        def row(r, c):
            a = bufA[r, pl.ds(0, 16)] * bufB[r, pl.ds(0, 16)]
            for v in range(1, 8):
                a = a + (bufA[r, pl.ds(v * 16, 16)]
                         * bufB[r, pl.ds(v * 16, 16)])
            outv[16 * j + (r >> 3), pl.ds((r & 7) * 16, 16)] = a
            return c
        lax.fori_loop(0, 128, row, 0)

    issue(0, bufA0, bufB0, semA0, semB0)

    def pair(t, c):
        j0 = 2 * t
        j1 = 2 * t + 1
        wait(j0, bufA0, bufB0, semA0, semB0)
        issue(j0 + 1, bufA1, bufB1, semA1, semB1)
        compute(j0, bufA0, bufB0)
        wait(j1, bufA1, bufB1, semA1, semB1)

        @pl.when(j1 + 1 < KC)
        def _():
            issue(j1 + 1, bufA0, bufB0, semA0, semB0)
        compute(j1, bufA1, bufB1)
        return c
    lax.fori_loop(0, KC // 2, pair, 0)
    wait(KC - 1, bufA0, bufB0, semA0, semB0)
    compute(KC - 1, bufA0, bufB0)
    pltpu.sync_copy(outv, out_hbm.at[pl.ds(wid * 16 * KC, 16 * KC)])


_cls_sc = pl.kernel(
    _cls_sc_body,
    out_type=[jax.ShapeDtypeStruct((NW * KC * 16, H), jnp.float32)],
    mesh=_mesh,
    scratch_types=[
        pltpu.VMEM((KC, 128), jnp.int32),
        pltpu.VMEM((KC, 128), jnp.int32),
        pltpu.VMEM((128, H), jnp.float32),
        pltpu.VMEM((128, H), jnp.float32),
        pltpu.VMEM((128, H), jnp.float32),
        pltpu.VMEM((128, H), jnp.float32),
        pltpu.VMEM((KC * 16, H), jnp.float32),
        pltpu.SemaphoreType.DMA,
        pltpu.SemaphoreType.DMA,
        pltpu.SemaphoreType.DMA,
        pltpu.SemaphoreType.DMA,
    ],
)


def _lanefold_body(x_ref, out_ref):
    i = lax.broadcasted_iota(jnp.int32, (H, H), 0)
    j = lax.broadcasted_iota(jnp.int32, (H, H), 1)
    m = ((i // 16) == j).astype(jnp.float32)
    out_ref[...] = jnp.dot(x_ref[...], m, preferred_element_type=jnp.float32)


_seg2 = pl.kernel(
    _seg2_body,
    out_type=[jax.ShapeDtypeStruct((NC, NPAD, H), jnp.float32)],
    mesh=_mesh,
    scratch_types=[
        pltpu.VMEM((SK, 128), jnp.int32),
        pltpu.VMEM((SK, 128), jnp.int32),
        pltpu.VMEM((128, H), jnp.float32),
        pltpu.VMEM((128, H), jnp.float32),
        pltpu.VMEM_SHARED((NPAD, H), jnp.float32),
        pltpu.SemaphoreType.DMA,
        pltpu.SemaphoreType.DMA,
        pltpu.SemaphoreType.DMA,
        pltpu.SemaphoreType.DMA,
    ],
)


ROW_BLK = 2048


def _dense1_body(cnt_p_ref, cnt_a_ref, xp_ref, s1_ref, w1wpl_ref, w1wpr_ref,
                 w1rwl_ref, w1rwr_ref, b1wp_ref, b1rw_ref, out_ref):
    cnt_p = cnt_p_ref[...]
    ind_p = (cnt_p > 0.0).astype(jnp.float32)
    w1sum = jnp.sum(w1wpl_ref[...], axis=0, keepdims=True)
    hp = ind_p * w1sum + jnp.dot(xp_ref[...], w1wpr_ref[...],
                                 preferred_element_type=jnp.float32)
    out_ref[1] = jnp.maximum(hp + b1wp_ref[...], 0.0)
    cnt_a = jnp.maximum(cnt_a_ref[...], 1.0)
    m_a = (s1_ref[0] + s1_ref[1]) / cnt_a
    c = jnp.sum(w1rwr_ref[...], axis=0, keepdims=True) + b1rw_ref[...]
    ha = jnp.dot(m_a, w1rwl_ref[...], preferred_element_type=jnp.float32) + c
    out_ref[0] = jnp.maximum(ha, 0.0)


def _dense2_body(cnt_p_ref, cnt_a_ref, h_ref, s2_ref, w2wpl_ref, w2wpr_ref,
                 w2rwl_ref, w2rwr_ref, b2wp_ref, b2rw_ref, out_ref):
    cnt_p = jnp.maximum(cnt_p_ref[...], 1.0)
    cnt_a = jnp.maximum(cnt_a_ref[...], 1.0)
    out_ref[1] = (jnp.dot(s2_ref[0] / cnt_p, w2wpl_ref[...],
                          preferred_element_type=jnp.float32)
                  + jnp.dot(h_ref[1], w2wpr_ref[...],
                            preferred_element_type=jnp.float32)
                  + b2wp_ref[...])
    out_ref[0] = (jnp.dot(s2_ref[1] / cnt_a, w2rwl_ref[...],
                          preferred_element_type=jnp.float32)
                  + jnp.dot(h_ref[0], w2rwr_ref[...],
                            preferred_element_type=jnp.float32)
                  + b2rw_ref[...])


def _row_spec():
    return pl.BlockSpec((ROW_BLK, H), lambda i: (i, 0))


def _pair_spec():
    return pl.BlockSpec((2, ROW_BLK, H), lambda i: (0, i, 0))


def _full_spec():
    return pl.BlockSpec((H, H), lambda i: (0, 0))


def _bias_spec():
    return pl.BlockSpec((1, H), lambda i: (0, 0))


def _cnt_spec():
    return pl.BlockSpec((ROW_BLK, 1), lambda i: (i, 0))


def _pad_idx(arr, target):
    pad = target - arr.shape[0]
    fill = N + (jnp.arange(pad, dtype=jnp.int32) % (NPAD - N))
    return jnp.concatenate([arr, fill])


def kernel(x_paper, x_author, edge_index_writes, edge_index_rev,
           edge_label_index, W1_wp_l, W1_wp_r, W1_rw_l, W1_rw_r, W2_wp_l,
           W2_wp_r, W2_rw_l, W2_rw_r, b1_wp, b1_rw, b2_wp, b2_rw):
    ew0 = edge_index_writes[0].astype(jnp.int32)
    ew1 = edge_index_writes[1].astype(jnp.int32)
    x_pad = jnp.pad(x_paper, ((0, NPAD - N), (0, 0)))

    g1 = _pad_idx(ew1, E1).reshape(NW, K1, 128)
    s1 = _pad_idx(ew0, E1).reshape(NW, K1, 128)
    s1p, cp_p, ca_p = _seg1(x_pad, g1, s1)
    cnt_p = (cp_p[0] + cp_p[1]).reshape(NPAD, 1)
    cnt_a = (ca_p[0] + ca_p[1]).reshape(NPAD, 1)

    grid = (NPAD // ROW_BLK,)
    hcat = pl.pallas_call(
        _dense1_body,
        grid=grid,
        in_specs=[_cnt_spec(), _cnt_spec(), _row_spec(), _pair_spec(),
                  _full_spec(), _full_spec(), _full_spec(), _full_spec(),
                  _bias_spec(), _bias_spec()],
        out_specs=_pair_spec(),
        out_shape=jax.ShapeDtypeStruct((2, NPAD, H), jnp.float32),
    )(cnt_p, cnt_a, x_pad, s1p, W1_wp_l, W1_wp_r, W1_rw_l, W1_rw_r,
      b1_wp[None, :], b1_rw[None, :])

    g2 = jnp.stack([
        _pad_idx(ew0, E2),
        _pad_idx(ew1, E2),
    ]).reshape(NC, NS, K2, 128)
    s2 = jnp.stack([
        _pad_idx(ew1, E2),
        _pad_idx(ew0, E2),
    ]).reshape(NC, NS, K2, 128)
    (s2out,) = _seg2(hcat, g2, s2)

    hcat2 = pl.pallas_call(
        _dense2_body,
        grid=grid,
        in_specs=[_cnt_spec(), _cnt_spec(), _pair_spec(), _pair_spec(),
                  _full_spec(), _full_spec(), _full_spec(), _full_spec(),
                  _bias_spec(), _bias_spec()],
        out_specs=_pair_spec(),
        out_shape=jax.ShapeDtypeStruct((2, NPAD, H), jnp.float32),
    )(cnt_p, cnt_a, hcat, s2out, W2_wp_l, W2_wp_r, W2_rw_l, W2_rw_r,
      b2_wp[None, :], b2_rw[None, :])

    el0 = edge_label_index[0].astype(jnp.int32)
    el1 = edge_label_index[1].astype(jnp.int32)
    ia = _pad_idx(el0, ELP).reshape(NW, KC, 128)
    ib = _pad_idx(el1, ELP).reshape(NW, KC, 128)
    (v,) = _cls_sc(hcat2, ia, ib)
    FB = 1664
    y = pl.pallas_call(
        _lanefold_body,
        grid=(v.shape[0] // FB,),
        in_specs=[pl.BlockSpec((FB, H), lambda i: (i, 0))],
        out_specs=pl.BlockSpec((FB, H), lambda i: (i, 0)),
        out_shape=jax.ShapeDtypeStruct((v.shape[0], H), jnp.float32),
    )(v)
    return y[:, :8].reshape(ELP)[:EL]
